# Initial kernel scaffold; baseline (speedup 1.0000x reference)
import jax
import jax.numpy as jnp
from jax.experimental import pallas as pl


def kernel(x, w1t, b1r, w2t, b2r):
    raise NotImplementedError("write your pallas kernel here")


# speedup vs baseline: 13.9611x; 13.9611x over previous
import jax
import jax.numpy as jnp
from jax.experimental import pallas as pl
from jax.experimental.pallas import tpu as pltpu

TILE_B = 512


def _mlp_fused_kernel(x_ref, w1_ref, b1_ref, w2_ref, b2_ref, o_ref):
    x = x_ref[...].astype(jnp.bfloat16)
    h = jnp.dot(x, w1_ref[...], preferred_element_type=jnp.float32)
    h = jnp.maximum(h + b1_ref[...], 0.0).astype(jnp.bfloat16)
    out = jnp.dot(h, w2_ref[...], preferred_element_type=jnp.float32)
    o_ref[...] = out + b2_ref[...]


@jax.jit
def kernel(x, w1t, b1r, w2t, b2r):
    B, I = x.shape
    H = w1t.shape[1]
    grid = (B // TILE_B,)

    w1b = w1t.astype(jnp.bfloat16)
    w2b = w2t.astype(jnp.bfloat16)

    flops = 4 * B * I * H
    bytes_accessed = 4 * (x.size + B * I) + 2 * (w1b.size + w2b.size)

    return pl.pallas_call(
        _mlp_fused_kernel,
        out_shape=jax.ShapeDtypeStruct((B, I), x.dtype),
        grid=grid,
        in_specs=[
            pl.BlockSpec((TILE_B, I), lambda i: (i, 0)),
            pl.BlockSpec((I, H), lambda i: (0, 0)),
            pl.BlockSpec((1, H), lambda i: (0, 0)),
            pl.BlockSpec((H, I), lambda i: (0, 0)),
            pl.BlockSpec((1, I), lambda i: (0, 0)),
        ],
        out_specs=pl.BlockSpec((TILE_B, I), lambda i: (i, 0)),
        compiler_params=pltpu.CompilerParams(
            dimension_semantics=("parallel",),
            vmem_limit_bytes=100 * 1024 * 1024,
        ),
        cost_estimate=pl.CostEstimate(
            flops=flops, transcendentals=0, bytes_accessed=bytes_accessed),
    )(x, w1b, b1r, w2b, b2r)


# speedup vs baseline: 14.0392x; 1.0056x over previous
import jax
import jax.numpy as jnp
from jax.experimental import pallas as pl
from jax.experimental.pallas import tpu as pltpu

TILE_B = 1024


def _mlp_fused_kernel(x_ref, w1_ref, b1_ref, w2_ref, b2_ref, o_ref):
    x = x_ref[...].astype(jnp.bfloat16)
    h = jnp.dot(x, w1_ref[...], preferred_element_type=jnp.float32)
    h = jnp.maximum(h + b1_ref[...], 0.0).astype(jnp.bfloat16)
    out = jnp.dot(h, w2_ref[...], preferred_element_type=jnp.float32)
    o_ref[...] = out + b2_ref[...]


@jax.jit
def kernel(x, w1t, b1r, w2t, b2r):
    B, I = x.shape
    H = w1t.shape[1]
    grid = (B // TILE_B,)

    w1b = w1t.astype(jnp.bfloat16)
    w2b = w2t.astype(jnp.bfloat16)

    flops = 4 * B * I * H
    bytes_accessed = 4 * (x.size + B * I) + 2 * (w1b.size + w2b.size)

    return pl.pallas_call(
        _mlp_fused_kernel,
        out_shape=jax.ShapeDtypeStruct((B, I), x.dtype),
        grid=grid,
        in_specs=[
            pl.BlockSpec((TILE_B, I), lambda i: (i, 0)),
            pl.BlockSpec((I, H), lambda i: (0, 0)),
            pl.BlockSpec((1, H), lambda i: (0, 0)),
            pl.BlockSpec((H, I), lambda i: (0, 0)),
            pl.BlockSpec((1, I), lambda i: (0, 0)),
        ],
        out_specs=pl.BlockSpec((TILE_B, I), lambda i: (i, 0)),
        compiler_params=pltpu.CompilerParams(
            dimension_semantics=("parallel",),
            vmem_limit_bytes=100 * 1024 * 1024,
        ),
        cost_estimate=pl.CostEstimate(
            flops=flops, transcendentals=0, bytes_accessed=bytes_accessed),
    )(x, w1b, b1r, w2b, b2r)


# speedup vs baseline: 14.3778x; 1.0241x over previous
import jax
import jax.numpy as jnp
from jax.experimental import pallas as pl
from jax.experimental.pallas import tpu as pltpu

TILE_B = 256


def _mlp_fused_kernel(x_ref, w1_ref, b1_ref, w2_ref, b2_ref, o_ref,
                      w1b_ref, w2b_ref):
    @pl.when(pl.program_id(0) == 0)
    def _cast_weights_once():
        w1b_ref[...] = w1_ref[...].astype(jnp.bfloat16)
        w2b_ref[...] = w2_ref[...].astype(jnp.bfloat16)

    x = x_ref[...].astype(jnp.bfloat16)
    h = jnp.dot(x, w1b_ref[...], preferred_element_type=jnp.float32)
    h = jnp.maximum(h + b1_ref[...], 0.0).astype(jnp.bfloat16)
    out = jnp.dot(h, w2b_ref[...], preferred_element_type=jnp.float32)
    o_ref[...] = out + b2_ref[...]


@jax.jit
def kernel(x, w1t, b1r, w2t, b2r):
    B, I = x.shape
    H = w1t.shape[1]
    grid = (B // TILE_B,)

    flops = 4 * B * I * H
    bytes_accessed = 4 * (x.size + B * I + w1t.size + w2t.size)

    return pl.pallas_call(
        _mlp_fused_kernel,
        out_shape=jax.ShapeDtypeStruct((B, I), x.dtype),
        grid=grid,
        in_specs=[
            pl.BlockSpec((TILE_B, I), lambda i: (i, 0)),
            pl.BlockSpec((I, H), lambda i: (0, 0)),
            pl.BlockSpec((1, H), lambda i: (0, 0)),
            pl.BlockSpec((H, I), lambda i: (0, 0)),
            pl.BlockSpec((1, I), lambda i: (0, 0)),
        ],
        out_specs=pl.BlockSpec((TILE_B, I), lambda i: (i, 0)),
        scratch_shapes=[
            pltpu.VMEM((I, H), jnp.bfloat16),
            pltpu.VMEM((H, I), jnp.bfloat16),
        ],
        compiler_params=pltpu.CompilerParams(
            dimension_semantics=("arbitrary",),
            vmem_limit_bytes=64 * 1024 * 1024,
        ),
        cost_estimate=pl.CostEstimate(
            flops=flops, transcendentals=0, bytes_accessed=bytes_accessed),
    )(x, w1t, b1r, w2t, b2r)


# speedup vs baseline: 14.5531x; 1.0122x over previous
import jax
import jax.numpy as jnp
from jax.experimental import pallas as pl
from jax.experimental.pallas import tpu as pltpu

TILE_B = 256


def _mlp_fused_kernel(x_ref, w1_ref, b1_ref, w2_ref, b2_ref, o_ref):
    h = jnp.dot(x_ref[...], w1_ref[...], preferred_element_type=jnp.float32)
    h = jnp.maximum(h + b1_ref[...], 0.0)
    out = jnp.dot(h, w2_ref[...], preferred_element_type=jnp.float32)
    o_ref[...] = out + b2_ref[...]


@jax.jit
def kernel(x, w1t, b1r, w2t, b2r):
    B, I = x.shape
    H = w1t.shape[1]
    grid = (B // TILE_B,)

    flops = 4 * B * I * H
    bytes_accessed = 4 * (x.size + B * I + w1t.size + w2t.size)

    return pl.pallas_call(
        _mlp_fused_kernel,
        out_shape=jax.ShapeDtypeStruct((B, I), x.dtype),
        grid=grid,
        in_specs=[
            pl.BlockSpec((TILE_B, I), lambda i: (i, 0)),
            pl.BlockSpec((I, H), lambda i: (0, 0)),
            pl.BlockSpec((1, H), lambda i: (0, 0)),
            pl.BlockSpec((H, I), lambda i: (0, 0)),
            pl.BlockSpec((1, I), lambda i: (0, 0)),
        ],
        out_specs=pl.BlockSpec((TILE_B, I), lambda i: (i, 0)),
        compiler_params=pltpu.CompilerParams(
            dimension_semantics=("arbitrary",),
            vmem_limit_bytes=64 * 1024 * 1024,
        ),
        cost_estimate=pl.CostEstimate(
            flops=flops, transcendentals=0, bytes_accessed=bytes_accessed),
    )(x, w1t, b1r, w2t, b2r)


# speedup vs baseline: 15.1167x; 1.0387x over previous
import jax
import jax.numpy as jnp
from jax.experimental import pallas as pl
from jax.experimental.pallas import tpu as pltpu

TILE_B = 512


def _mlp_fused_kernel(x_ref, w1_ref, b1_ref, w2_ref, b2_ref, o_ref):
    h = jnp.dot(x_ref[...], w1_ref[...], preferred_element_type=jnp.float32)
    h = jnp.maximum(h + b1_ref[...], 0.0)
    out = jnp.dot(h, w2_ref[...], preferred_element_type=jnp.float32)
    o_ref[...] = out + b2_ref[...]


@jax.jit
def kernel(x, w1t, b1r, w2t, b2r):
    B, I = x.shape
    H = w1t.shape[1]
    grid = (B // TILE_B,)

    flops = 4 * B * I * H
    bytes_accessed = 4 * (x.size + B * I + w1t.size + w2t.size)

    return pl.pallas_call(
        _mlp_fused_kernel,
        out_shape=jax.ShapeDtypeStruct((B, I), x.dtype),
        grid=grid,
        in_specs=[
            pl.BlockSpec((TILE_B, I), lambda i: (i, 0)),
            pl.BlockSpec((I, H), lambda i: (0, 0)),
            pl.BlockSpec((1, H), lambda i: (0, 0)),
            pl.BlockSpec((H, I), lambda i: (0, 0)),
            pl.BlockSpec((1, I), lambda i: (0, 0)),
        ],
        out_specs=pl.BlockSpec((TILE_B, I), lambda i: (i, 0)),
        compiler_params=pltpu.CompilerParams(
            dimension_semantics=("arbitrary",),
            vmem_limit_bytes=64 * 1024 * 1024,
        ),
        cost_estimate=pl.CostEstimate(
            flops=flops, transcendentals=0, bytes_accessed=bytes_accessed),
    )(x, w1t, b1r, w2t, b2r)


# speedup vs baseline: 15.2251x; 1.0072x over previous
import jax
import jax.numpy as jnp
from jax.experimental import pallas as pl
from jax.experimental.pallas import tpu as pltpu

TILE_B = 1024


def _mlp_fused_kernel(x_ref, w1_ref, b1_ref, w2_ref, b2_ref, o_ref):
    h = jnp.dot(x_ref[...], w1_ref[...], preferred_element_type=jnp.float32)
    h = jnp.maximum(h + b1_ref[...], 0.0)
    out = jnp.dot(h, w2_ref[...], preferred_element_type=jnp.float32)
    o_ref[...] = out + b2_ref[...]


@jax.jit
def kernel(x, w1t, b1r, w2t, b2r):
    B, I = x.shape
    H = w1t.shape[1]
    grid = (B // TILE_B,)

    flops = 4 * B * I * H
    bytes_accessed = 4 * (x.size + B * I + w1t.size + w2t.size)

    return pl.pallas_call(
        _mlp_fused_kernel,
        out_shape=jax.ShapeDtypeStruct((B, I), x.dtype),
        grid=grid,
        in_specs=[
            pl.BlockSpec((TILE_B, I), lambda i: (i, 0)),
            pl.BlockSpec((I, H), lambda i: (0, 0)),
            pl.BlockSpec((1, H), lambda i: (0, 0)),
            pl.BlockSpec((H, I), lambda i: (0, 0)),
            pl.BlockSpec((1, I), lambda i: (0, 0)),
        ],
        out_specs=pl.BlockSpec((TILE_B, I), lambda i: (i, 0)),
        compiler_params=pltpu.CompilerParams(
            dimension_semantics=("arbitrary",),
            vmem_limit_bytes=64 * 1024 * 1024,
        ),
        cost_estimate=pl.CostEstimate(
            flops=flops, transcendentals=0, bytes_accessed=bytes_accessed),
    )(x, w1t, b1r, w2t, b2r)


# speedup vs baseline: 15.3766x; 1.0100x over previous
import jax
import jax.numpy as jnp
from jax.experimental import pallas as pl
from jax.experimental.pallas import tpu as pltpu

TILE_B = 1024


def _mlp_fused_kernel(x_ref, w1_hbm, b1_ref, w2_hbm, b2_ref, o_ref,
                      w1_v, w2_v, sems):
    j = pl.program_id(0)

    @pl.when(j == 0)
    def _fetch_weights():
        pltpu.make_async_copy(w1_hbm, w1_v, sems.at[0]).start()
        pltpu.make_async_copy(w2_hbm, w2_v, sems.at[1]).start()
        pltpu.make_async_copy(w1_hbm, w1_v, sems.at[0]).wait()

    h = jnp.dot(x_ref[...], w1_v[...], preferred_element_type=jnp.float32)
    h = jnp.maximum(h + b1_ref[...], 0.0)

    @pl.when(j == 0)
    def _wait_w2():
        pltpu.make_async_copy(w2_hbm, w2_v, sems.at[1]).wait()

    out = jnp.dot(h, w2_v[...], preferred_element_type=jnp.float32)
    o_ref[...] = out + b2_ref[...]


@jax.jit
def kernel(x, w1t, b1r, w2t, b2r):
    B, I = x.shape
    H = w1t.shape[1]
    grid = (B // TILE_B,)

    flops = 4 * B * I * H
    bytes_accessed = 4 * (x.size + B * I + w1t.size + w2t.size)

    return pl.pallas_call(
        _mlp_fused_kernel,
        out_shape=jax.ShapeDtypeStruct((B, I), x.dtype),
        grid=grid,
        in_specs=[
            pl.BlockSpec((TILE_B, I), lambda i: (i, 0)),
            pl.BlockSpec(memory_space=pl.ANY),
            pl.BlockSpec((1, H), lambda i: (0, 0)),
            pl.BlockSpec(memory_space=pl.ANY),
            pl.BlockSpec((1, I), lambda i: (0, 0)),
        ],
        out_specs=pl.BlockSpec((TILE_B, I), lambda i: (i, 0)),
        scratch_shapes=[
            pltpu.VMEM((I, H), jnp.float32),
            pltpu.VMEM((H, I), jnp.float32),
            pltpu.SemaphoreType.DMA((2,)),
        ],
        compiler_params=pltpu.CompilerParams(
            dimension_semantics=("arbitrary",),
            vmem_limit_bytes=64 * 1024 * 1024,
        ),
        cost_estimate=pl.CostEstimate(
            flops=flops, transcendentals=0, bytes_accessed=bytes_accessed),
    )(x, w1t, b1r, w2t, b2r)
